# Initial kernel scaffold; baseline (speedup 1.0000x reference)
import jax
import jax.numpy as jnp
from jax.experimental import pallas as pl


def kernel(w_ih_l0, w_hh_l0, b_ih_l0, b_hh_l0, w_ih_l0_r, w_hh_l0_r, b_ih_l0_r, b_hh_l0_r, w_ih_l1, w_hh_l1, b_ih_l1, b_hh_l1, w_ih_l1_r, w_hh_l1_r, b_ih_l1_r, b_hh_l1_r, fc1, fc2, x):
    raise NotImplementedError("write your pallas kernel here")


# speedup vs baseline: 1.2756x; 1.2756x over previous
import functools

import jax
import jax.numpy as jnp
from jax import lax
from jax.experimental import pallas as pl
from jax.experimental.pallas import tpu as pltpu


def _pool_body(x_ref, o_ref, *, inv_hw):
    o_ref[...] = jnp.sum(x_ref[...], axis=-1, keepdims=True) * inv_hw


def _rnn_mlp_body(pmT_ref, w0f_ref, w0b_ref, b0_ref, whh0T_ref,
                  wih1fT_ref, whh1fT_ref, b1f_ref, wih1bT_ref, b1b_ref,
                  w1L_ref, fc2L_ref, eg_ref, egT_ref, gates_ref,
                  h0_scr, c1f_scr, *, D, N, hid, mid):
    f32 = jnp.float32
    H2 = 2 * hid

    def l0_step(t, carry):
        h, c = carry
        xf = pmT_ref[pl.ds(t, 1), :]
        xb = pmT_ref[pl.ds(D - 1 - t, 1), :]
        pre = (w0f_ref[...] * xf + w0b_ref[...] * xb + b0_ref[...]
               + jnp.dot(whh0T_ref[...], h, preferred_element_type=f32))
        i = jax.nn.sigmoid(pre[0 * H2:1 * H2])
        f = jax.nn.sigmoid(pre[1 * H2:2 * H2])
        g = jnp.tanh(pre[2 * H2:3 * H2])
        o = jax.nn.sigmoid(pre[3 * H2:4 * H2])
        c = f * c + i * g
        h = o * jnp.tanh(c)
        col_f = pl.multiple_of(t * N, N)
        col_b = pl.multiple_of((D - 1 - t) * N, N)
        h0_scr[0:hid, pl.ds(col_f, N)] = h[0:hid]
        h0_scr[hid:H2, pl.ds(col_b, N)] = h[hid:H2]
        return h, c

    z0 = jnp.zeros((H2, N), f32)
    lax.fori_loop(0, D, l0_step, (z0, z0))

    c1f_scr[...] = jnp.dot(wih1fT_ref[...], h0_scr[...],
                           preferred_element_type=f32)

    def l1_step(t, carry):
        h, c = carry
        col = pl.multiple_of(t * N, N)
        pre = (c1f_scr[:, pl.ds(col, N)] + b1f_ref[...]
               + jnp.dot(whh1fT_ref[...], h, preferred_element_type=f32))
        i = jax.nn.sigmoid(pre[0 * hid:1 * hid])
        f = jax.nn.sigmoid(pre[1 * hid:2 * hid])
        g = jnp.tanh(pre[2 * hid:3 * hid])
        o = jax.nn.sigmoid(pre[3 * hid:4 * hid])
        c = f * c + i * g
        h = o * jnp.tanh(c)
        return h, c

    z1 = jnp.zeros((hid, N), f32)
    h1f, _ = lax.fori_loop(0, D, l1_step, (z1, z1))

    col_last = pl.multiple_of((D - 1) * N, N)
    pre_b = (jnp.dot(wih1bT_ref[...], h0_scr[:, pl.ds(col_last, N)],
                     preferred_element_type=f32) + b1b_ref[...])
    ib = jax.nn.sigmoid(pre_b[0 * hid:1 * hid])
    gb = jnp.tanh(pre_b[2 * hid:3 * hid])
    ob = jax.nn.sigmoid(pre_b[3 * hid:4 * hid])
    h1b = ob * jnp.tanh(ib * gb)

    feat = jnp.concatenate([h1f, h1b], axis=0)

    zrows = []
    for m in range(mid):
        zrows.append(jnp.sum(w1L_ref[m] * feat, axis=0, keepdims=True))
    zrow = jnp.concatenate(zrows, axis=0)
    z = jnp.maximum(jnp.dot(zrow, eg_ref[...],
                            preferred_element_type=f32), 0.0)
    zexp = jnp.dot(z, egT_ref[...], preferred_element_type=f32)
    grow = jax.nn.sigmoid(jnp.sum(fc2L_ref[...] * zexp, axis=0,
                                  keepdims=True))
    gates_ref[...] = grow


def _apply_body(x_ref, g_ref, o_ref):
    o_ref[...] = x_ref[...] * g_ref[...]


def _interleave_dirs(a_f, a_b, hid):
    sh = a_f.shape[1:]
    st = jnp.stack([a_f.reshape((4, hid) + sh), a_b.reshape((4, hid) + sh)],
                   axis=1)
    return st.reshape((8 * hid,) + sh)


def kernel(w_ih_l0, w_hh_l0, b_ih_l0, b_hh_l0,
           w_ih_l0_r, w_hh_l0_r, b_ih_l0_r, b_hh_l0_r,
           w_ih_l1, w_hh_l1, b_ih_l1, b_hh_l1,
           w_ih_l1_r, w_hh_l1_r, b_ih_l1_r, b_hh_l1_r,
           fc1, fc2, x):
    f32 = jnp.float32
    B, C, D, HS, WS = x.shape
    HW = HS * WS
    N = B * C
    hid = w_hh_l0.shape[1]
    H2 = 2 * hid
    G = 4 * hid
    mid = fc1.shape[0]

    R = N * D
    xp = x.reshape(R, HW)
    TR = 1024 if R % 1024 == 0 else R
    pooled = pl.pallas_call(
        functools.partial(_pool_body, inv_hw=1.0 / float(HW)),
        out_shape=jax.ShapeDtypeStruct((R, 1), f32),
        grid=(R // TR,),
        in_specs=[pl.BlockSpec((TR, HW), lambda i: (i, 0))],
        out_specs=pl.BlockSpec((TR, 1), lambda i: (i, 0)),
        compiler_params=pltpu.CompilerParams(
            dimension_semantics=("parallel",)),
    )(xp)

    pmT = pooled.reshape(N, D).T

    zg = jnp.zeros((G,), f32)
    w0f = _interleave_dirs(w_ih_l0[:, 0], zg, hid).reshape(2 * G, 1)
    w0b = _interleave_dirs(zg, w_ih_l0_r[:, 0], hid).reshape(2 * G, 1)
    b0 = _interleave_dirs(b_ih_l0 + b_hh_l0,
                          b_ih_l0_r + b_hh_l0_r, hid).reshape(2 * G, 1)
    zh = jnp.zeros((G, hid), f32)
    whh0T = jnp.concatenate(
        [_interleave_dirs(w_hh_l0, zh, hid),
         _interleave_dirs(zh, w_hh_l0_r, hid)], axis=1)

    wih1fT = w_ih_l1
    whh1fT = w_hh_l1
    b1f = (b_ih_l1 + b_hh_l1).reshape(G, 1)
    wih1bT = w_ih_l1_r
    b1b = (b_ih_l1_r + b_hh_l1_r).reshape(G, 1)

    w1L = jnp.tile(fc1.reshape(mid, C, H2).transpose(0, 2, 1), (1, 1, B))
    fc2L = jnp.tile(fc2.T, (1, B))
    eg = (jnp.arange(N)[:, None] // C ==
          jnp.arange(B)[None, :]).astype(f32)
    egT = eg.T

    args2 = (pmT, w0f, w0b, b0, whh0T, wih1fT, whh1fT, b1f, wih1bT, b1b,
             w1L, fc2L, eg, egT)
    body2 = functools.partial(_rnn_mlp_body, D=D, N=N, hid=hid, mid=mid)
    grow = pl.pallas_call(
        body2,
        out_shape=jax.ShapeDtypeStruct((1, N), f32),
        grid=(1,),
        in_specs=[pl.BlockSpec(a.shape, functools.partial(
            lambda nd, i: (0,) * nd, a.ndim)) for a in args2],
        out_specs=pl.BlockSpec((1, N), lambda i: (0, 0)),
        scratch_shapes=[pltpu.VMEM((H2, D * N), f32),
                        pltpu.VMEM((G, D * N), f32)],
        compiler_params=pltpu.CompilerParams(
            dimension_semantics=("arbitrary",)),
    )(*args2)
    gates = grow.reshape(N, 1)

    L = D * HW
    xa = x.reshape(N, L)
    TL = 2048 if L % 2048 == 0 else L
    out = pl.pallas_call(
        _apply_body,
        out_shape=jax.ShapeDtypeStruct((N, L), x.dtype),
        grid=(L // TL,),
        in_specs=[pl.BlockSpec((N, TL), lambda j: (0, j)),
                  pl.BlockSpec((N, 1), lambda j: (0, 0))],
        out_specs=pl.BlockSpec((N, TL), lambda j: (0, j)),
        compiler_params=pltpu.CompilerParams(
            dimension_semantics=("parallel",)),
    )(xa, gates)
    return out.reshape(B, C, D, HS, WS)


# speedup vs baseline: 1.3352x; 1.0467x over previous
import functools

import jax
import jax.numpy as jnp
from jax import lax
from jax.experimental import pallas as pl
from jax.experimental.pallas import tpu as pltpu


def _pool_body(x_ref, o_ref, *, inv_hw):
    o_ref[...] = jnp.sum(x_ref[0], axis=(-2, -1)) * inv_hw


def _rnn_mlp_body(pm_ref, w0f_ref, w0b_ref, b0_ref, whh0T_ref,
                  wih1fT_ref, whh1fT_ref, b1f_ref, wih1bT_ref, b1b_ref,
                  w1L_ref, fc2L_ref, eg_ref, egT_ref, gates_ref,
                  pmT_ref, h0_scr, c1f_scr, *, D, N, hid, mid):
    f32 = jnp.float32
    H2 = 2 * hid
    pmT_ref[...] = pm_ref[...].T

    def l0_step(t, carry):
        h, c = carry
        xf = pmT_ref[pl.ds(t, 1), :]
        xb = pmT_ref[pl.ds(D - 1 - t, 1), :]
        pre = (w0f_ref[...] * xf + w0b_ref[...] * xb + b0_ref[...]
               + jnp.dot(whh0T_ref[...], h, preferred_element_type=f32))
        i = jax.nn.sigmoid(pre[0 * H2:1 * H2])
        f = jax.nn.sigmoid(pre[1 * H2:2 * H2])
        g = jnp.tanh(pre[2 * H2:3 * H2])
        o = jax.nn.sigmoid(pre[3 * H2:4 * H2])
        c = f * c + i * g
        h = o * jnp.tanh(c)
        col_f = pl.multiple_of(t * N, N)
        col_b = pl.multiple_of((D - 1 - t) * N, N)
        h0_scr[0:hid, pl.ds(col_f, N)] = h[0:hid]
        h0_scr[hid:H2, pl.ds(col_b, N)] = h[hid:H2]
        return h, c

    z0 = jnp.zeros((H2, N), f32)
    lax.fori_loop(0, D, l0_step, (z0, z0))

    c1f_scr[...] = jnp.dot(wih1fT_ref[...], h0_scr[...],
                           preferred_element_type=f32)

    def l1_step(t, carry):
        h, c = carry
        col = pl.multiple_of(t * N, N)
        pre = (c1f_scr[:, pl.ds(col, N)] + b1f_ref[...]
               + jnp.dot(whh1fT_ref[...], h, preferred_element_type=f32))
        i = jax.nn.sigmoid(pre[0 * hid:1 * hid])
        f = jax.nn.sigmoid(pre[1 * hid:2 * hid])
        g = jnp.tanh(pre[2 * hid:3 * hid])
        o = jax.nn.sigmoid(pre[3 * hid:4 * hid])
        c = f * c + i * g
        h = o * jnp.tanh(c)
        return h, c

    z1 = jnp.zeros((hid, N), f32)
    h1f, _ = lax.fori_loop(0, D, l1_step, (z1, z1))

    col_last = pl.multiple_of((D - 1) * N, N)
    pre_b = (jnp.dot(wih1bT_ref[...], h0_scr[:, pl.ds(col_last, N)],
                     preferred_element_type=f32) + b1b_ref[...])
    ib = jax.nn.sigmoid(pre_b[0 * hid:1 * hid])
    gb = jnp.tanh(pre_b[2 * hid:3 * hid])
    ob = jax.nn.sigmoid(pre_b[3 * hid:4 * hid])
    h1b = ob * jnp.tanh(ib * gb)

    feat = jnp.concatenate([h1f, h1b], axis=0)

    zrows = []
    for m in range(mid):
        zrows.append(jnp.sum(w1L_ref[m] * feat, axis=0, keepdims=True))
    zrow = jnp.concatenate(zrows, axis=0)
    z = jnp.maximum(jnp.dot(zrow, eg_ref[...],
                            preferred_element_type=f32), 0.0)
    zexp = jnp.dot(z, egT_ref[...], preferred_element_type=f32)
    grow = jax.nn.sigmoid(jnp.sum(fc2L_ref[...] * zexp, axis=0,
                                  keepdims=True))
    gates_ref[...] = grow


def _apply_body(x_ref, g_ref, o_ref):
    g = g_ref[...].reshape(1, g_ref.shape[0], 1, 1, 1)
    o_ref[...] = x_ref[...] * g


def _interleave_dirs(a_f, a_b, hid):
    sh = a_f.shape[1:]
    st = jnp.stack([a_f.reshape((4, hid) + sh), a_b.reshape((4, hid) + sh)],
                   axis=1)
    return st.reshape((8 * hid,) + sh)


def kernel(w_ih_l0, w_hh_l0, b_ih_l0, b_hh_l0,
           w_ih_l0_r, w_hh_l0_r, b_ih_l0_r, b_hh_l0_r,
           w_ih_l1, w_hh_l1, b_ih_l1, b_hh_l1,
           w_ih_l1_r, w_hh_l1_r, b_ih_l1_r, b_hh_l1_r,
           fc1, fc2, x):
    f32 = jnp.float32
    B, C, D, HS, WS = x.shape
    HW = HS * WS
    N = B * C
    hid = w_hh_l0.shape[1]
    H2 = 2 * hid
    G = 4 * hid
    mid = fc1.shape[0]

    CT = 16 if C % 16 == 0 else C
    NC = C // CT
    pooled = pl.pallas_call(
        functools.partial(_pool_body, inv_hw=1.0 / float(HW)),
        out_shape=jax.ShapeDtypeStruct((N, D), f32),
        grid=(B, NC),
        in_specs=[pl.BlockSpec((1, CT, D, HS, WS),
                               lambda b, j: (b, j, 0, 0, 0))],
        out_specs=pl.BlockSpec((CT, D),
                               functools.partial(
                                   lambda nc, b, j: (b * nc + j, 0), NC)),
        compiler_params=pltpu.CompilerParams(
            dimension_semantics=("parallel", "parallel")),
    )(x)

    zg = jnp.zeros((G,), f32)
    w0f = _interleave_dirs(w_ih_l0[:, 0], zg, hid).reshape(2 * G, 1)
    w0b = _interleave_dirs(zg, w_ih_l0_r[:, 0], hid).reshape(2 * G, 1)
    b0 = _interleave_dirs(b_ih_l0 + b_hh_l0,
                          b_ih_l0_r + b_hh_l0_r, hid).reshape(2 * G, 1)
    zh = jnp.zeros((G, hid), f32)
    whh0T = jnp.concatenate(
        [_interleave_dirs(w_hh_l0, zh, hid),
         _interleave_dirs(zh, w_hh_l0_r, hid)], axis=1)

    wih1fT = w_ih_l1
    whh1fT = w_hh_l1
    b1f = (b_ih_l1 + b_hh_l1).reshape(G, 1)
    wih1bT = w_ih_l1_r
    b1b = (b_ih_l1_r + b_hh_l1_r).reshape(G, 1)

    w1L = jnp.tile(fc1.reshape(mid, C, H2).transpose(0, 2, 1), (1, 1, B))
    fc2L = jnp.tile(fc2.T, (1, B))
    eg = (jnp.arange(N)[:, None] // C ==
          jnp.arange(B)[None, :]).astype(f32)
    egT = eg.T

    args2 = (pooled, w0f, w0b, b0, whh0T, wih1fT, whh1fT, b1f, wih1bT, b1b,
             w1L, fc2L, eg, egT)
    body2 = functools.partial(_rnn_mlp_body, D=D, N=N, hid=hid, mid=mid)
    grow = pl.pallas_call(
        body2,
        out_shape=jax.ShapeDtypeStruct((1, N), f32),
        grid=(1,),
        in_specs=[pl.BlockSpec(a.shape, functools.partial(
            lambda nd, i: (0,) * nd, a.ndim)) for a in args2],
        out_specs=pl.BlockSpec((1, N), lambda i: (0, 0)),
        scratch_shapes=[pltpu.VMEM((D, N), f32),
                        pltpu.VMEM((H2, D * N), f32),
                        pltpu.VMEM((G, D * N), f32)],
        compiler_params=pltpu.CompilerParams(
            dimension_semantics=("arbitrary",)),
    )(*args2)
    gates = grow.reshape(N, 1)

    out = pl.pallas_call(
        _apply_body,
        out_shape=jax.ShapeDtypeStruct((B, C, D, HS, WS), x.dtype),
        grid=(B, NC),
        in_specs=[pl.BlockSpec((1, CT, D, HS, WS),
                               lambda b, j: (b, j, 0, 0, 0)),
                  pl.BlockSpec((CT, 1),
                               functools.partial(
                                   lambda nc, b, j: (b * nc + j, 0), NC))],
        out_specs=pl.BlockSpec((1, CT, D, HS, WS),
                               lambda b, j: (b, j, 0, 0, 0)),
        compiler_params=pltpu.CompilerParams(
            dimension_semantics=("parallel", "parallel")),
    )(x, gates)
    return out


# speedup vs baseline: 5.3182x; 3.9832x over previous
import functools

import jax
import jax.numpy as jnp
from jax import lax
from jax.experimental import pallas as pl
from jax.experimental.pallas import tpu as pltpu


def _pool_body(x_ref, o_ref, *, inv_hw):
    s = jnp.sum(x_ref[0], axis=(1, 2)) * inv_hw
    o_ref[...] = s.T


def _rnn_mlp_body(pm_ref, w0f_ref, w0b_ref, b0_ref, whh0T_ref,
                  wih1fT_ref, whh1fT_ref, b1f_ref, wih1bT_ref, b1b_ref,
                  w1L_ref, fc2L_ref, eg_ref, egT_ref, gates_ref,
                  pmT_ref, h0_scr, c1f_scr, *, D, N, hid, mid):
    f32 = jnp.float32
    H2 = 2 * hid
    pmT_ref[...] = pm_ref[...].T

    def l0_step(t, carry):
        h, c = carry
        xf = pmT_ref[pl.ds(t, 1), :]
        xb = pmT_ref[pl.ds(D - 1 - t, 1), :]
        pre = (w0f_ref[...] * xf + w0b_ref[...] * xb + b0_ref[...]
               + jnp.dot(whh0T_ref[...], h, preferred_element_type=f32))
        i = jax.nn.sigmoid(pre[0 * H2:1 * H2])
        f = jax.nn.sigmoid(pre[1 * H2:2 * H2])
        g = jnp.tanh(pre[2 * H2:3 * H2])
        o = jax.nn.sigmoid(pre[3 * H2:4 * H2])
        c = f * c + i * g
        h = o * jnp.tanh(c)
        col_f = pl.multiple_of(t * N, N)
        col_b = pl.multiple_of((D - 1 - t) * N, N)
        h0_scr[0:hid, pl.ds(col_f, N)] = h[0:hid]
        h0_scr[hid:H2, pl.ds(col_b, N)] = h[hid:H2]
        return h, c

    z0 = jnp.zeros((H2, N), f32)
    lax.fori_loop(0, D, l0_step, (z0, z0))

    c1f_scr[...] = jnp.dot(wih1fT_ref[...], h0_scr[...],
                           preferred_element_type=f32)

    def l1_step(t, carry):
        h, c = carry
        col = pl.multiple_of(t * N, N)
        pre = (c1f_scr[:, pl.ds(col, N)] + b1f_ref[...]
               + jnp.dot(whh1fT_ref[...], h, preferred_element_type=f32))
        i = jax.nn.sigmoid(pre[0 * hid:1 * hid])
        f = jax.nn.sigmoid(pre[1 * hid:2 * hid])
        g = jnp.tanh(pre[2 * hid:3 * hid])
        o = jax.nn.sigmoid(pre[3 * hid:4 * hid])
        c = f * c + i * g
        h = o * jnp.tanh(c)
        return h, c

    z1 = jnp.zeros((hid, N), f32)
    h1f, _ = lax.fori_loop(0, D, l1_step, (z1, z1))

    col_last = pl.multiple_of((D - 1) * N, N)
    pre_b = (jnp.dot(wih1bT_ref[...], h0_scr[:, pl.ds(col_last, N)],
                     preferred_element_type=f32) + b1b_ref[...])
    ib = jax.nn.sigmoid(pre_b[0 * hid:1 * hid])
    gb = jnp.tanh(pre_b[2 * hid:3 * hid])
    ob = jax.nn.sigmoid(pre_b[3 * hid:4 * hid])
    h1b = ob * jnp.tanh(ib * gb)

    feat = jnp.concatenate([h1f, h1b], axis=0)

    zrows = []
    for m in range(mid):
        zrows.append(jnp.sum(w1L_ref[m] * feat, axis=0, keepdims=True))
    zrow = jnp.concatenate(zrows, axis=0)
    z = jnp.maximum(jnp.dot(zrow, eg_ref[...],
                            preferred_element_type=f32), 0.0)
    zexp = jnp.dot(z, egT_ref[...], preferred_element_type=f32)
    grow = jax.nn.sigmoid(jnp.sum(fc2L_ref[...] * zexp, axis=0,
                                  keepdims=True))
    gates_ref[...] = grow


def _apply_body(x_ref, g_ref, o_ref):
    g = g_ref[...].reshape(1, 1, 1, 1, g_ref.shape[-1])
    o_ref[...] = x_ref[...] * g


def _interleave_dirs(a_f, a_b, hid):
    sh = a_f.shape[1:]
    st = jnp.stack([a_f.reshape((4, hid) + sh), a_b.reshape((4, hid) + sh)],
                   axis=1)
    return st.reshape((8 * hid,) + sh)


def kernel(w_ih_l0, w_hh_l0, b_ih_l0, b_hh_l0,
           w_ih_l0_r, w_hh_l0_r, b_ih_l0_r, b_hh_l0_r,
           w_ih_l1, w_hh_l1, b_ih_l1, b_hh_l1,
           w_ih_l1_r, w_hh_l1_r, b_ih_l1_r, b_hh_l1_r,
           fc1, fc2, x):
    f32 = jnp.float32
    B, C, D, HS, WS = x.shape
    HW = HS * WS
    N = B * C
    hid = w_hh_l0.shape[1]
    H2 = 2 * hid
    G = 4 * hid
    mid = fc1.shape[0]

    xt = jnp.transpose(x, (0, 2, 3, 4, 1))
    pooled = pl.pallas_call(
        functools.partial(_pool_body, inv_hw=1.0 / float(HW)),
        out_shape=jax.ShapeDtypeStruct((N, D), f32),
        grid=(B,),
        in_specs=[pl.BlockSpec((1, D, HS, WS, C),
                               lambda b: (b, 0, 0, 0, 0))],
        out_specs=pl.BlockSpec((C, D), lambda b: (b, 0)),
        compiler_params=pltpu.CompilerParams(
            dimension_semantics=("parallel",)),
    )(xt)

    zg = jnp.zeros((G,), f32)
    w0f = _interleave_dirs(w_ih_l0[:, 0], zg, hid).reshape(2 * G, 1)
    w0b = _interleave_dirs(zg, w_ih_l0_r[:, 0], hid).reshape(2 * G, 1)
    b0 = _interleave_dirs(b_ih_l0 + b_hh_l0,
                          b_ih_l0_r + b_hh_l0_r, hid).reshape(2 * G, 1)
    zh = jnp.zeros((G, hid), f32)
    whh0T = jnp.concatenate(
        [_interleave_dirs(w_hh_l0, zh, hid),
         _interleave_dirs(zh, w_hh_l0_r, hid)], axis=1)

    wih1fT = w_ih_l1
    whh1fT = w_hh_l1
    b1f = (b_ih_l1 + b_hh_l1).reshape(G, 1)
    wih1bT = w_ih_l1_r
    b1b = (b_ih_l1_r + b_hh_l1_r).reshape(G, 1)

    w1L = jnp.tile(fc1.reshape(mid, C, H2).transpose(0, 2, 1), (1, 1, B))
    fc2L = jnp.tile(fc2.T, (1, B))
    eg = (jnp.arange(N)[:, None] // C ==
          jnp.arange(B)[None, :]).astype(f32)
    egT = eg.T

    args2 = (pooled, w0f, w0b, b0, whh0T, wih1fT, whh1fT, b1f, wih1bT, b1b,
             w1L, fc2L, eg, egT)
    body2 = functools.partial(_rnn_mlp_body, D=D, N=N, hid=hid, mid=mid)
    grow = pl.pallas_call(
        body2,
        out_shape=jax.ShapeDtypeStruct((1, N), f32),
        grid=(1,),
        in_specs=[pl.BlockSpec(a.shape, functools.partial(
            lambda nd, i: (0,) * nd, a.ndim)) for a in args2],
        out_specs=pl.BlockSpec((1, N), lambda i: (0, 0)),
        scratch_shapes=[pltpu.VMEM((D, N), f32),
                        pltpu.VMEM((H2, D * N), f32),
                        pltpu.VMEM((G, D * N), f32)],
        compiler_params=pltpu.CompilerParams(
            dimension_semantics=("arbitrary",)),
    )(*args2)
    gates = grow.reshape(B, 1, C)

    out_t = pl.pallas_call(
        _apply_body,
        out_shape=jax.ShapeDtypeStruct((B, D, HS, WS, C), x.dtype),
        grid=(B,),
        in_specs=[pl.BlockSpec((1, D, HS, WS, C),
                               lambda b: (b, 0, 0, 0, 0)),
                  pl.BlockSpec((1, 1, C), lambda b: (b, 0, 0))],
        out_specs=pl.BlockSpec((1, D, HS, WS, C),
                               lambda b: (b, 0, 0, 0, 0)),
        compiler_params=pltpu.CompilerParams(
            dimension_semantics=("parallel",)),
    )(xt, gates)
    return jnp.transpose(out_t, (0, 4, 1, 2, 3))


# speedup vs baseline: 6.3427x; 1.1926x over previous
import functools

import jax
import jax.numpy as jnp
from jax import lax
from jax.experimental import pallas as pl
from jax.experimental.pallas import tpu as pltpu


def _dot0(a, b):
    return lax.dot_general(a, b, (((0,), (0,)), ((), ())),
                           preferred_element_type=jnp.float32)


def _pool_rnn_body(x_ref, wi0f_ref, wi0b_ref, bi0f_ref, bh0f_ref, bi0b_ref,
                   bh0b_ref, wh0f_ref, wh0b_ref, wi1f_ref, wh1f_ref,
                   bi1f_ref, bh1f_ref, wi1b_ref, bi1b_ref, bh1b_ref,
                   w1L_ref, fc2L_ref, eg_ref, egT_ref, gates_ref,
                   pm3_scr, pmT_scr, a0_scr, asm_scr, h0_scr, c1f_scr,
                   *, B, C, D, N, hid, mid, inv_hw):
    f32 = jnp.float32
    H2 = 2 * hid
    b = pl.program_id(0)
    pm3_scr[b] = jnp.sum(x_ref[0], axis=(1, 2)) * inv_hw

    @pl.when(b == B - 1)
    def _rnn():
        for bb in range(B):
            pmT_scr[:, bb * C:(bb + 1) * C] = pm3_scr[bb]

        a0_scr[...] = jnp.zeros(a0_scr.shape, f32)
        asm_scr[...] = jnp.zeros(asm_scr.shape, f32)
        for q in range(4):
            cf = slice(q * H2, q * H2 + hid)
            cb = slice(q * H2 + hid, q * H2 + H2)
            g = slice(q * hid, (q + 1) * hid)
            a0_scr[0:hid, cf] = wh0f_ref[:, g]
            a0_scr[hid:H2, cb] = wh0b_ref[:, g]
            asm_scr[0:1, cf] = wi0f_ref[:, g]
            asm_scr[1:2, cb] = wi0b_ref[:, g]
            asm_scr[2:3, cf] = bi0f_ref[:, g] + bh0f_ref[:, g]
            asm_scr[2:3, cb] = bi0b_ref[:, g] + bh0b_ref[:, g]
        asm_scr[3:4, 0:4 * hid] = bi1f_ref[...] + bh1f_ref[...]
        asm_scr[4:5, 0:4 * hid] = bi1b_ref[...] + bh1b_ref[...]
        cols = asm_scr[...].T
        w0f = cols[:, 0:1]
        w0b = cols[:, 1:2]
        b0c = cols[:, 2:3]
        b1f = cols[0:4 * hid, 3:4]
        b1b = cols[0:4 * hid, 4:5]

        def l0_step(t, carry):
            h, c = carry
            xf = pmT_scr[pl.ds(t, 1), :]
            xb = pmT_scr[pl.ds(D - 1 - t, 1), :]
            pre = w0f * xf + w0b * xb + b0c + _dot0(a0_scr[...], h)
            i = jax.nn.sigmoid(pre[0 * H2:1 * H2])
            f = jax.nn.sigmoid(pre[1 * H2:2 * H2])
            g = jnp.tanh(pre[2 * H2:3 * H2])
            o = jax.nn.sigmoid(pre[3 * H2:4 * H2])
            c = f * c + i * g
            h = o * jnp.tanh(c)
            col_f = pl.multiple_of(t * N, N)
            col_b = pl.multiple_of((D - 1 - t) * N, N)
            h0_scr[0:hid, pl.ds(col_f, N)] = h[0:hid]
            h0_scr[hid:H2, pl.ds(col_b, N)] = h[hid:H2]
            return h, c

        z0 = jnp.zeros((H2, N), f32)
        lax.fori_loop(0, D, l0_step, (z0, z0))

        c1f_scr[...] = _dot0(wi1f_ref[...], h0_scr[...])

        def l1_step(t, carry):
            h, c = carry
            col = pl.multiple_of(t * N, N)
            pre = (c1f_scr[:, pl.ds(col, N)] + b1f
                   + _dot0(wh1f_ref[...], h))
            i = jax.nn.sigmoid(pre[0 * hid:1 * hid])
            f = jax.nn.sigmoid(pre[1 * hid:2 * hid])
            g = jnp.tanh(pre[2 * hid:3 * hid])
            o = jax.nn.sigmoid(pre[3 * hid:4 * hid])
            c = f * c + i * g
            h = o * jnp.tanh(c)
            return h, c

        z1 = jnp.zeros((hid, N), f32)
        h1f, _ = lax.fori_loop(0, D, l1_step, (z1, z1))

        col_last = pl.multiple_of((D - 1) * N, N)
        pre_b = _dot0(wi1b_ref[...], h0_scr[:, pl.ds(col_last, N)]) + b1b
        ib = jax.nn.sigmoid(pre_b[0 * hid:1 * hid])
        gb = jnp.tanh(pre_b[2 * hid:3 * hid])
        ob = jax.nn.sigmoid(pre_b[3 * hid:4 * hid])
        h1b = ob * jnp.tanh(ib * gb)

        feat = jnp.concatenate([h1f, h1b], axis=0)

        zrows = []
        for m in range(mid):
            zrows.append(jnp.sum(w1L_ref[m] * feat, axis=0, keepdims=True))
        zrow = jnp.concatenate(zrows, axis=0)
        z = jnp.maximum(jnp.dot(zrow, eg_ref[...],
                                preferred_element_type=f32), 0.0)
        zexp = jnp.dot(z, egT_ref[...], preferred_element_type=f32)
        grow = jax.nn.sigmoid(jnp.sum(fc2L_ref[...] * zexp, axis=0,
                                      keepdims=True))
        gates_ref[...] = grow


def _apply_body(x_ref, g_ref, o_ref):
    g = g_ref[...].reshape(1, 1, 1, 1, g_ref.shape[-1])
    o_ref[...] = x_ref[...] * g


def kernel(w_ih_l0, w_hh_l0, b_ih_l0, b_hh_l0,
           w_ih_l0_r, w_hh_l0_r, b_ih_l0_r, b_hh_l0_r,
           w_ih_l1, w_hh_l1, b_ih_l1, b_hh_l1,
           w_ih_l1_r, w_hh_l1_r, b_ih_l1_r, b_hh_l1_r,
           fc1, fc2, x):
    f32 = jnp.float32
    B, C, D, HS, WS = x.shape
    HW = HS * WS
    N = B * C
    hid = w_hh_l0.shape[1]
    H2 = 2 * hid
    G = 4 * hid
    mid = fc1.shape[0]

    xt = jnp.transpose(x, (0, 2, 3, 4, 1))

    wi0f = w_ih_l0.T
    wi0b = w_ih_l0_r.T
    bi0f = b_ih_l0.reshape(1, G)
    bh0f = b_hh_l0.reshape(1, G)
    bi0b = b_ih_l0_r.reshape(1, G)
    bh0b = b_hh_l0_r.reshape(1, G)
    wh0f = w_hh_l0.T
    wh0b = w_hh_l0_r.T
    wi1f = w_ih_l1.T
    wh1f = w_hh_l1.T
    bi1f = b_ih_l1.reshape(1, G)
    bh1f = b_hh_l1.reshape(1, G)
    wi1b = w_ih_l1_r.T
    bi1b = b_ih_l1_r.reshape(1, G)
    bh1b = b_hh_l1_r.reshape(1, G)

    w1L = jnp.tile(fc1.reshape(mid, C, H2).transpose(0, 2, 1), (1, 1, B))
    fc2L = jnp.tile(fc2.T, (1, B))
    eg = (jnp.arange(N)[:, None] // C ==
          jnp.arange(B)[None, :]).astype(f32)
    egT = eg.T

    wargs = (wi0f, wi0b, bi0f, bh0f, bi0b, bh0b, wh0f, wh0b, wi1f, wh1f,
             bi1f, bh1f, wi1b, bi1b, bh1b, w1L, fc2L, eg, egT)
    body12 = functools.partial(_pool_rnn_body, B=B, C=C, D=D, N=N, hid=hid,
                               mid=mid, inv_hw=1.0 / float(HW))
    grow = pl.pallas_call(
        body12,
        out_shape=jax.ShapeDtypeStruct((1, N), f32),
        grid=(B,),
        in_specs=[pl.BlockSpec((1, D, HS, WS, C),
                               lambda b: (b, 0, 0, 0, 0))] +
                 [pl.BlockSpec(a.shape, functools.partial(
                     lambda nd, b: (0,) * nd, a.ndim)) for a in wargs],
        out_specs=pl.BlockSpec((1, N), lambda b: (0, 0)),
        scratch_shapes=[pltpu.VMEM((B, D, C), f32),
                        pltpu.VMEM((D, N), f32),
                        pltpu.VMEM((H2, 2 * G), f32),
                        pltpu.VMEM((8, 2 * G), f32),
                        pltpu.VMEM((H2, D * N), f32),
                        pltpu.VMEM((G, D * N), f32)],
        compiler_params=pltpu.CompilerParams(
            dimension_semantics=("arbitrary",)),
    )(xt, *wargs)
    gates = grow.reshape(B, 1, C)

    out_t = pl.pallas_call(
        _apply_body,
        out_shape=jax.ShapeDtypeStruct((B, D, HS, WS, C), x.dtype),
        grid=(B,),
        in_specs=[pl.BlockSpec((1, D, HS, WS, C),
                               lambda b: (b, 0, 0, 0, 0)),
                  pl.BlockSpec((1, 1, C), lambda b: (b, 0, 0))],
        out_specs=pl.BlockSpec((1, D, HS, WS, C),
                               lambda b: (b, 0, 0, 0, 0)),
        compiler_params=pltpu.CompilerParams(
            dimension_semantics=("parallel",)),
    )(xt, gates)
    return jnp.transpose(out_t, (0, 4, 1, 2, 3))


# speedup vs baseline: 6.7639x; 1.0664x over previous
import functools

import jax
import jax.numpy as jnp
from jax import lax
from jax.experimental import pallas as pl
from jax.experimental.pallas import tpu as pltpu


def _dot0(a, b):
    return lax.dot_general(a, b, (((0,), (0,)), ((), ())),
                           preferred_element_type=jnp.float32)


def _pool_rnn_body(x_ref, wi0f_ref, wi0b_ref, bi0f_ref, bh0f_ref, bi0b_ref,
                   bh0b_ref, wh0f_ref, wh0b_ref, wi1f_ref, wh1f_ref,
                   bi1f_ref, bh1f_ref, wi1b_ref, bi1b_ref, bh1b_ref,
                   w1L_ref, fc2T_ref, gates_ref,
                   pm3_scr, pmT_scr, a0_scr, asm_scr, h0_scr, c1f_scr,
                   *, B, C, D, N, hid, mid, inv_hw):
    f32 = jnp.float32
    H2 = 2 * hid
    b = pl.program_id(0)
    pm3_scr[b] = jnp.sum(x_ref[0], axis=(1, 2)) * inv_hw

    @pl.when(b == B - 1)
    def _rnn():
        for bb in range(B):
            pmT_scr[:, bb * C:(bb + 1) * C] = pm3_scr[bb]

        a0_scr[...] = jnp.zeros(a0_scr.shape, f32)
        asm_scr[...] = jnp.zeros(asm_scr.shape, f32)
        for q in range(4):
            cf = slice(q * H2, q * H2 + hid)
            cb = slice(q * H2 + hid, q * H2 + H2)
            g = slice(q * hid, (q + 1) * hid)
            a0_scr[0:hid, cf] = wh0f_ref[:, g]
            a0_scr[hid:H2, cb] = wh0b_ref[:, g]
            asm_scr[0:1, cf] = wi0f_ref[:, g]
            asm_scr[1:2, cb] = wi0b_ref[:, g]
            asm_scr[2:3, cf] = bi0f_ref[:, g] + bh0f_ref[:, g]
            asm_scr[2:3, cb] = bi0b_ref[:, g] + bh0b_ref[:, g]
        asm_scr[3:4, 0:4 * hid] = bi1f_ref[...] + bh1f_ref[...]
        asm_scr[4:5, 0:4 * hid] = bi1b_ref[...] + bh1b_ref[...]
        cols = asm_scr[...].T
        w0f = cols[:, 0:1]
        w0b = cols[:, 1:2]
        b0c = cols[:, 2:3]
        b1f = cols[0:4 * hid, 3:4]
        b1b = cols[0:4 * hid, 4:5]

        def l0_step(t, carry):
            h, c = carry
            xf = pmT_scr[pl.ds(t, 1), :]
            xb = pmT_scr[pl.ds(D - 1 - t, 1), :]
            pre = w0f * xf + w0b * xb + b0c + _dot0(a0_scr[...], h)
            i = jax.nn.sigmoid(pre[0 * H2:1 * H2])
            f = jax.nn.sigmoid(pre[1 * H2:2 * H2])
            g = jnp.tanh(pre[2 * H2:3 * H2])
            o = jax.nn.sigmoid(pre[3 * H2:4 * H2])
            c = f * c + i * g
            h = o * jnp.tanh(c)
            col_f = pl.multiple_of(t * N, N)
            col_b = pl.multiple_of((D - 1 - t) * N, N)
            h0_scr[0:hid, pl.ds(col_f, N)] = h[0:hid]
            h0_scr[hid:H2, pl.ds(col_b, N)] = h[hid:H2]
            return h, c

        z0 = jnp.zeros((H2, N), f32)
        lax.fori_loop(0, D, l0_step, (z0, z0), unroll=4)

        c1f_scr[...] = _dot0(wi1f_ref[...], h0_scr[...])

        def l1_step(t, carry):
            h, c = carry
            col = pl.multiple_of(t * N, N)
            pre = (c1f_scr[:, pl.ds(col, N)] + b1f
                   + _dot0(wh1f_ref[...], h))
            i = jax.nn.sigmoid(pre[0 * hid:1 * hid])
            f = jax.nn.sigmoid(pre[1 * hid:2 * hid])
            g = jnp.tanh(pre[2 * hid:3 * hid])
            o = jax.nn.sigmoid(pre[3 * hid:4 * hid])
            c = f * c + i * g
            h = o * jnp.tanh(c)
            return h, c

        z1 = jnp.zeros((hid, N), f32)
        h1f, _ = lax.fori_loop(0, D, l1_step, (z1, z1), unroll=4)

        col_last = pl.multiple_of((D - 1) * N, N)
        pre_b = _dot0(wi1b_ref[...], h0_scr[:, pl.ds(col_last, N)]) + b1b
        ib = jax.nn.sigmoid(pre_b[0 * hid:1 * hid])
        gb = jnp.tanh(pre_b[2 * hid:3 * hid])
        ob = jax.nn.sigmoid(pre_b[3 * hid:4 * hid])
        h1b = ob * jnp.tanh(ib * gb)

        feat = jnp.concatenate([h1f, h1b], axis=0)

        i32 = jnp.int32
        eg = (lax.broadcasted_iota(i32, (N, B), 0) // C ==
              lax.broadcasted_iota(i32, (N, B), 1)).astype(f32)
        egT = (lax.broadcasted_iota(i32, (B, N), 1) // C ==
               lax.broadcasted_iota(i32, (B, N), 0)).astype(f32)
        fc2L = jnp.concatenate([fc2T_ref[...]] * B, axis=1)
        zrows = []
        for m in range(mid):
            zrows.append(jnp.sum(w1L_ref[m] * feat, axis=0, keepdims=True))
        zrow = jnp.concatenate(zrows, axis=0)
        z = jnp.maximum(jnp.dot(zrow, eg,
                                preferred_element_type=f32), 0.0)
        zexp = jnp.dot(z, egT, preferred_element_type=f32)
        grow = jax.nn.sigmoid(jnp.sum(fc2L * zexp, axis=0,
                                      keepdims=True))
        gates_ref[...] = grow


def _apply_body(x_ref, g_ref, o_ref):
    g = g_ref[...].reshape(1, 1, 1, 1, g_ref.shape[-1])
    o_ref[...] = x_ref[...] * g


def kernel(w_ih_l0, w_hh_l0, b_ih_l0, b_hh_l0,
           w_ih_l0_r, w_hh_l0_r, b_ih_l0_r, b_hh_l0_r,
           w_ih_l1, w_hh_l1, b_ih_l1, b_hh_l1,
           w_ih_l1_r, w_hh_l1_r, b_ih_l1_r, b_hh_l1_r,
           fc1, fc2, x):
    f32 = jnp.float32
    B, C, D, HS, WS = x.shape
    HW = HS * WS
    N = B * C
    hid = w_hh_l0.shape[1]
    H2 = 2 * hid
    G = 4 * hid
    mid = fc1.shape[0]

    xt = jnp.transpose(x, (0, 2, 3, 4, 1))

    wi0f = w_ih_l0.T
    wi0b = w_ih_l0_r.T
    bi0f = b_ih_l0.reshape(1, G)
    bh0f = b_hh_l0.reshape(1, G)
    bi0b = b_ih_l0_r.reshape(1, G)
    bh0b = b_hh_l0_r.reshape(1, G)
    wh0f = w_hh_l0.T
    wh0b = w_hh_l0_r.T
    wi1f = w_ih_l1.T
    wh1f = w_hh_l1.T
    bi1f = b_ih_l1.reshape(1, G)
    bh1f = b_hh_l1.reshape(1, G)
    wi1b = w_ih_l1_r.T
    bi1b = b_ih_l1_r.reshape(1, G)
    bh1b = b_hh_l1_r.reshape(1, G)

    w1L = jnp.tile(fc1.reshape(mid, C, H2).transpose(0, 2, 1), (1, 1, B))
    fc2T = fc2.T

    wargs = (wi0f, wi0b, bi0f, bh0f, bi0b, bh0b, wh0f, wh0b, wi1f, wh1f,
             bi1f, bh1f, wi1b, bi1b, bh1b, w1L, fc2T)
    body12 = functools.partial(_pool_rnn_body, B=B, C=C, D=D, N=N, hid=hid,
                               mid=mid, inv_hw=1.0 / float(HW))
    grow = pl.pallas_call(
        body12,
        out_shape=jax.ShapeDtypeStruct((1, N), f32),
        grid=(B,),
        in_specs=[pl.BlockSpec((1, D, HS, WS, C),
                               lambda b: (b, 0, 0, 0, 0))] +
                 [pl.BlockSpec(a.shape, functools.partial(
                     lambda nd, b: (0,) * nd, a.ndim)) for a in wargs],
        out_specs=pl.BlockSpec((1, N), lambda b: (0, 0)),
        scratch_shapes=[pltpu.VMEM((B, D, C), f32),
                        pltpu.VMEM((D, N), f32),
                        pltpu.VMEM((H2, 2 * G), f32),
                        pltpu.VMEM((8, 2 * G), f32),
                        pltpu.VMEM((H2, D * N), f32),
                        pltpu.VMEM((G, D * N), f32)],
        compiler_params=pltpu.CompilerParams(
            dimension_semantics=("arbitrary",)),
    )(xt, *wargs)
    gates = grow.reshape(B, 1, C)

    out_t = pl.pallas_call(
        _apply_body,
        out_shape=jax.ShapeDtypeStruct((B, D, HS, WS, C), x.dtype),
        grid=(B,),
        in_specs=[pl.BlockSpec((1, D, HS, WS, C),
                               lambda b: (b, 0, 0, 0, 0)),
                  pl.BlockSpec((1, 1, C), lambda b: (b, 0, 0))],
        out_specs=pl.BlockSpec((1, D, HS, WS, C),
                               lambda b: (b, 0, 0, 0, 0)),
        compiler_params=pltpu.CompilerParams(
            dimension_semantics=("parallel",)),
    )(xt, gates)
    return jnp.transpose(out_t, (0, 4, 1, 2, 3))
